# baseline (speedup 1.0000x reference)
import jax
import jax.numpy as jnp
from jax import lax
from jax.experimental import pallas as pl
from jax.experimental.pallas import tpu as pltpu
from jax.experimental.pallas import tpu_sc as plsc

_T = 26
_V = 100000
_D = 64
_B = 4096
_NC = 2
_NS = 16
_NW = _NC * _NS
_VB = (_V + 127) // 128
_VP2 = _VB * 64


def _ta_body(tabT_ref, out_ref):
    blk = tabT_ref[0]
    tr = jnp.transpose(blk)
    q = lax.broadcasted_iota(jnp.int32, (_D, 128), 0)
    c = lax.broadcasted_iota(jnp.int32, (_D, 128), 1)
    se = (c == 2 * q).astype(jnp.float32)
    so = (c == 2 * q + 1).astype(jnp.float32)
    ev = jax.lax.dot(se, tr, precision=jax.lax.Precision.HIGHEST,
                     preferred_element_type=jnp.float32)
    od = jax.lax.dot(so, tr, precision=jax.lax.Precision.HIGHEST,
                     preferred_element_type=jnp.float32)
    out_ref[0] = jnp.concatenate([ev, od], axis=1)


_ta = pl.pallas_call(
    _ta_body,
    grid=(_T, _VB),
    in_specs=[pl.BlockSpec((1, _D, 128), lambda t, c: (t, 0, c))],
    out_specs=pl.BlockSpec((1, _D, 128), lambda t, c: (t, c, 0)),
    out_shape=jax.ShapeDtypeStruct((_T, _VP2, 2 * _D), jnp.float32),
)


_CHUNK = _B // _NW
_K = 4
_G = 3


def _gbody(idx_hbm, pairs_hbm, out_hbm, idx_v, pidx_v, rows_v, gsem, wsem):
    wid = lax.axis_index("s") * _NC + lax.axis_index("c")
    b0 = wid * _CHUNK
    pltpu.sync_copy(idx_hbm.at[:, pl.ds(b0, _CHUNK)], idx_v)

    def prep(t):
        def pr(i, _):
            v = idx_v[t, pl.ds(i * 16, 16)]
            pidx_v[t & (_K - 1), pl.ds(i * 16, 16)] = (
                lax.shift_right_logical(v, 1))
            return 0

        lax.fori_loop(0, _CHUNK // 16, pr, 0)
        pltpu.async_copy(
            pairs_hbm.at[t].at[pidx_v.at[t & (_K - 1)]],
            rows_v.at[t & (_K - 1)], gsem)

    for t in range(_G):
        prep(t)

    def ch(t, _):
        s = t & (_K - 1)
        pltpu.make_async_copy(
            pairs_hbm.at[0].at[pidx_v.at[s]], rows_v.at[s], gsem).wait()
        pltpu.async_copy(
            rows_v.at[s], out_hbm.at[t, pl.ds(b0, _CHUNK), :], wsem)

        @pl.when(t + _G < _T)
        def _():
            @pl.when(t >= 1)
            def _():
                pltpu.make_async_copy(
                    rows_v.at[(t - 1) & (_K - 1)],
                    out_hbm.at[0, pl.ds(0, _CHUNK), :], wsem).wait()

            prep(t + _G)

        return 0

    lax.fori_loop(0, _T, ch, 0)

    def dr(t, _):
        pltpu.make_async_copy(
            rows_v.at[t & (_K - 1)],
            out_hbm.at[0, pl.ds(0, _CHUNK), :], wsem).wait()
        return 0

    lax.fori_loop(_T - _K, _T, dr, 0)


_mesh = plsc.VectorSubcoreMesh(core_axis_name="c", subcore_axis_name="s")

_gather = pl.kernel(
    _gbody,
    out_type=jax.ShapeDtypeStruct((_T, _B, 2 * _D), jnp.float32),
    mesh=_mesh,
    scratch_types=[
        pltpu.VMEM((_T, _CHUNK), jnp.int32),
        pltpu.VMEM((_K, _CHUNK), jnp.int32),
        pltpu.VMEM((_K, _CHUNK, 2 * _D), jnp.float32),
        pltpu.SemaphoreType.DMA,
        pltpu.SemaphoreType.DMA,
    ],
    compiler_params=pltpu.CompilerParams(
        use_tc_tiling_on_sc=True, needs_layout_passes=False),
)


def _tc_body(g_ref, idx_ref, out_ref):
    gb = g_ref[0]
    h = idx_ref[0, 0] & 1
    sel = jnp.where(h[:, None] == 1, gb[:, _D:], gb[:, :_D])
    out_ref[0] = jnp.transpose(sel)


_tc = pl.pallas_call(
    _tc_body,
    grid=(_T, _B // 128),
    in_specs=[
        pl.BlockSpec((1, 128, 2 * _D), lambda t, c: (t, c, 0)),
        pl.BlockSpec((1, 1, 128), lambda t, c: (t * (_B // 128) + c, 0, 0)),
    ],
    out_specs=pl.BlockSpec((1, _D, 128), lambda t, c: (t, 0, c)),
    out_shape=jax.ShapeDtypeStruct((_T, _D, _B), jnp.float32),
)


@jax.jit
def kernel(inputs, tables):
    tabT = jnp.transpose(tables, (0, 2, 1))
    idx = jnp.transpose(inputs).astype(jnp.int32)
    pairs = _ta(tabT)
    g = _gather(idx, pairs)
    out = _tc(g, idx.reshape(_T * (_B // 128), 1, 128))
    return out.transpose(2, 0, 1)


# speedup vs baseline: 5.3883x; 5.3883x over previous
import jax
import jax.numpy as jnp
from jax import lax
from jax.experimental import pallas as pl
from jax.experimental.pallas import tpu as pltpu
from jax.experimental.pallas import tpu_sc as plsc

_T = 26
_V = 100000
_D = 64
_B = 4096
_NC = 2
_NS = 16
_NW = _NC * _NS
_VB = (_V + 127) // 128
_VP2 = _VB * 64


_W = 5888
_NSTEP = (_VB * 128) // _W


def _ta_body(tabT_ref, out_ref):
    q = lax.broadcasted_iota(jnp.int32, (_D, 128), 0)
    c = lax.broadcasted_iota(jnp.int32, (_D, 128), 1)
    se = (c == 2 * q).astype(jnp.float32)
    so = (c == 2 * q + 1).astype(jnp.float32)
    for j in range(_W // 128):
        blk = tabT_ref[0, :, j * 128:(j + 1) * 128]
        tr = jnp.transpose(blk)
        ev = jax.lax.dot(se, tr, precision=jax.lax.Precision.HIGHEST,
                         preferred_element_type=jnp.float32)
        od = jax.lax.dot(so, tr, precision=jax.lax.Precision.HIGHEST,
                         preferred_element_type=jnp.float32)
        out_ref[0, j * 64:(j + 1) * 64, :] = jnp.concatenate(
            [ev, od], axis=1)


_ta = pl.pallas_call(
    _ta_body,
    grid=(_T, _NSTEP),
    in_specs=[pl.BlockSpec((1, _D, _W), lambda t, c: (t, 0, c))],
    out_specs=pl.BlockSpec((1, _W // 2, 128), lambda t, c: (t, c, 0)),
    out_shape=jax.ShapeDtypeStruct((_T, _VP2, 2 * _D), jnp.float32),
)


_CHUNK = _B // _NW
_K = 4
_G = 3


def _gbody(idx_hbm, pairs_hbm, out_hbm, idx_v, pidx_v, rows_v, gsem, wsem):
    wid = lax.axis_index("s") * _NC + lax.axis_index("c")
    b0 = wid * _CHUNK
    pltpu.sync_copy(idx_hbm.at[:, pl.ds(b0, _CHUNK)], idx_v)

    def prep(t):
        def pr(i, _):
            v = idx_v[t, pl.ds(i * 16, 16)]
            pidx_v[t & (_K - 1), pl.ds(i * 16, 16)] = (
                lax.shift_right_logical(v, 1))
            return 0

        lax.fori_loop(0, _CHUNK // 16, pr, 0)
        pltpu.async_copy(
            pairs_hbm.at[t].at[pidx_v.at[t & (_K - 1)]],
            rows_v.at[t & (_K - 1)], gsem)

    for t in range(_G):
        prep(t)

    def ch(t, _):
        s = t & (_K - 1)
        pltpu.make_async_copy(
            pairs_hbm.at[0].at[pidx_v.at[s]], rows_v.at[s], gsem).wait()
        pltpu.async_copy(
            rows_v.at[s], out_hbm.at[t, pl.ds(b0, _CHUNK), :], wsem)

        @pl.when(t + _G < _T)
        def _():
            @pl.when(t >= 1)
            def _():
                pltpu.make_async_copy(
                    rows_v.at[(t - 1) & (_K - 1)],
                    out_hbm.at[0, pl.ds(0, _CHUNK), :], wsem).wait()

            prep(t + _G)

        return 0

    lax.fori_loop(0, _T, ch, 0)

    def dr(t, _):
        pltpu.make_async_copy(
            rows_v.at[t & (_K - 1)],
            out_hbm.at[0, pl.ds(0, _CHUNK), :], wsem).wait()
        return 0

    lax.fori_loop(_T - _K, _T, dr, 0)


_mesh = plsc.VectorSubcoreMesh(core_axis_name="c", subcore_axis_name="s")

_gather = pl.kernel(
    _gbody,
    out_type=jax.ShapeDtypeStruct((_T, _B, 2 * _D), jnp.float32),
    mesh=_mesh,
    scratch_types=[
        pltpu.VMEM((_T, _CHUNK), jnp.int32),
        pltpu.VMEM((_K, _CHUNK), jnp.int32),
        pltpu.VMEM((_K, _CHUNK, 2 * _D), jnp.float32),
        pltpu.SemaphoreType.DMA,
        pltpu.SemaphoreType.DMA,
    ],
    compiler_params=pltpu.CompilerParams(
        use_tc_tiling_on_sc=True, needs_layout_passes=False),
)


_CB = 512


def _tc_body(g_ref, idx_ref, out_ref):
    gb = g_ref[0]
    h = idx_ref[0, 0] & 1
    sel = jnp.where(h[:, None] == 1, gb[:, _D:], gb[:, :_D])
    out_ref[0] = jnp.transpose(sel)


_tc = pl.pallas_call(
    _tc_body,
    grid=(_T, _B // _CB),
    in_specs=[
        pl.BlockSpec((1, _CB, 2 * _D), lambda t, c: (t, c, 0)),
        pl.BlockSpec((1, 1, _CB), lambda t, c: (t * (_B // _CB) + c, 0, 0)),
    ],
    out_specs=pl.BlockSpec((1, _D, _CB), lambda t, c: (t, 0, c)),
    out_shape=jax.ShapeDtypeStruct((_T, _D, _B), jnp.float32),
)


@jax.jit
def kernel(inputs, tables):
    tabT = jnp.transpose(tables, (0, 2, 1))
    idx = jnp.transpose(inputs).astype(jnp.int32)
    pairs = _ta(tabT)
    g = _gather(idx, pairs)
    out = _tc(g, idx.reshape(_T * (_B // _CB), 1, _CB))
    return out.transpose(2, 0, 1)


# speedup vs baseline: 9.3330x; 1.7321x over previous
import jax
import jax.numpy as jnp
from jax import lax
from jax.experimental import pallas as pl
from jax.experimental.pallas import tpu as pltpu
from jax.experimental.pallas import tpu_sc as plsc

_T = 26
_V = 100000
_D = 64
_B = 4096
_NC = 2
_NS = 16
_NW = _NC * _NS
_VB = (_V + 127) // 128
_VP2 = _VB * 64


_W = 5888
_NSTEP = (_VB * 128) // _W


def _ta_body(tabT_ref, out_ref):
    p = lax.broadcasted_iota(jnp.int32, (128, 128), 0)
    c = lax.broadcasted_iota(jnp.int32, (128, 128), 1)
    tgt = jnp.where(p < _D, 2 * p, 2 * (p - _D) + 1)
    perm = (c == tgt).astype(jnp.float32)
    for j in range(_W // 128):
        blk = tabT_ref[0, :, j * 128:(j + 1) * 128]
        oc = lax.dot_general(
            perm, blk, dimension_numbers=(((1,), (1,)), ((), ())),
            precision=jax.lax.Precision.HIGHEST,
            preferred_element_type=jnp.float32)
        out_ref[0, j * 64:(j + 1) * 64, 0:_D] = oc[0:_D]
        out_ref[0, j * 64:(j + 1) * 64, _D:2 * _D] = oc[_D:128]


_ta = pl.pallas_call(
    _ta_body,
    grid=(_T, _NSTEP),
    in_specs=[pl.BlockSpec((1, _D, _W), lambda t, c: (t, 0, c))],
    out_specs=pl.BlockSpec((1, _W // 2, 128), lambda t, c: (t, c, 0)),
    out_shape=jax.ShapeDtypeStruct((_T, _VP2, 2 * _D), jnp.float32),
)


_CHUNK = _B // _NW
_K = 4
_G = 3


def _gbody(idx_hbm, pairs_hbm, out_hbm, idx_v, pidx_v, rows_v, gsem, wsem):
    wid = lax.axis_index("s") * _NC + lax.axis_index("c")
    b0 = wid * _CHUNK
    pltpu.sync_copy(idx_hbm.at[:, pl.ds(b0, _CHUNK)], idx_v)

    def prep(t):
        def pr(i, _):
            v = idx_v[t, pl.ds(i * 16, 16)]
            pidx_v[t & (_K - 1), pl.ds(i * 16, 16)] = (
                lax.shift_right_logical(v, 1))
            return 0

        lax.fori_loop(0, _CHUNK // 16, pr, 0)
        pltpu.async_copy(
            pairs_hbm.at[t].at[pidx_v.at[t & (_K - 1)]],
            rows_v.at[t & (_K - 1)], gsem)

    for t in range(_G):
        prep(t)

    def ch(t, _):
        s = t & (_K - 1)
        pltpu.make_async_copy(
            pairs_hbm.at[0].at[pidx_v.at[s]], rows_v.at[s], gsem).wait()
        pltpu.async_copy(
            rows_v.at[s], out_hbm.at[t, pl.ds(b0, _CHUNK), :], wsem)

        @pl.when(t + _G < _T)
        def _():
            @pl.when(t >= 1)
            def _():
                pltpu.make_async_copy(
                    rows_v.at[(t - 1) & (_K - 1)],
                    out_hbm.at[0, pl.ds(0, _CHUNK), :], wsem).wait()

            prep(t + _G)

        return 0

    lax.fori_loop(0, _T, ch, 0)

    def dr(t, _):
        pltpu.make_async_copy(
            rows_v.at[t & (_K - 1)],
            out_hbm.at[0, pl.ds(0, _CHUNK), :], wsem).wait()
        return 0

    lax.fori_loop(_T - _K, _T, dr, 0)


_mesh = plsc.VectorSubcoreMesh(core_axis_name="c", subcore_axis_name="s")

_gather = pl.kernel(
    _gbody,
    out_type=jax.ShapeDtypeStruct((_T, _B, 2 * _D), jnp.float32),
    mesh=_mesh,
    scratch_types=[
        pltpu.VMEM((_T, _CHUNK), jnp.int32),
        pltpu.VMEM((_K, _CHUNK), jnp.int32),
        pltpu.VMEM((_K, _CHUNK, 2 * _D), jnp.float32),
        pltpu.SemaphoreType.DMA,
        pltpu.SemaphoreType.DMA,
    ],
    compiler_params=pltpu.CompilerParams(
        use_tc_tiling_on_sc=True, needs_layout_passes=False),
)


_CB = 512


def _tc_body(g_ref, idx_ref, out_ref):
    gb = g_ref[0]
    h = idx_ref[0, 0] & 1
    sel = jnp.where(h[:, None] == 1, gb[:, _D:], gb[:, :_D])
    out_ref[0] = jnp.transpose(sel)


_tc = pl.pallas_call(
    _tc_body,
    grid=(_T, _B // _CB),
    in_specs=[
        pl.BlockSpec((1, _CB, 2 * _D), lambda t, c: (t, c, 0)),
        pl.BlockSpec((1, 1, _CB), lambda t, c: (t * (_B // _CB) + c, 0, 0)),
    ],
    out_specs=pl.BlockSpec((1, _D, _CB), lambda t, c: (t, 0, c)),
    out_shape=jax.ShapeDtypeStruct((_T, _D, _B), jnp.float32),
)


@jax.jit
def kernel(inputs, tables):
    tabT = jnp.transpose(tables, (0, 2, 1))
    idx = jnp.transpose(inputs).astype(jnp.int32)
    pairs = _ta(tabT)
    g = _gather(idx, pairs)
    out = _tc(g, idx.reshape(_T * (_B // _CB), 1, _CB))
    return out.transpose(2, 0, 1)


# speedup vs baseline: 13.5667x; 1.4536x over previous
import jax
import jax.numpy as jnp
from jax import lax
from jax.experimental import pallas as pl
from jax.experimental.pallas import tpu as pltpu
from jax.experimental.pallas import tpu_sc as plsc

_T = 26
_V = 100000
_D = 64
_B = 4096
_NC = 2
_NS = 16
_NW = _NC * _NS
_VB = (_V + 127) // 128
_VP2 = _VB * 64


_W = 5888
_NSTEP = (_VB * 128) // _W


def _ta_body(tabT_ref, out_ref):
    p = lax.broadcasted_iota(jnp.int32, (128, 128), 0)
    c = lax.broadcasted_iota(jnp.int32, (128, 128), 1)
    tgt = jnp.where(p < _D, 2 * p, 2 * (p - _D) + 1)
    perm = (c == tgt).astype(jnp.float32)
    for j in range(0, _W // 128, 2):
        blk2 = jnp.concatenate(
            [tabT_ref[0, :, j * 128:(j + 1) * 128],
             tabT_ref[0, :, (j + 1) * 128:(j + 2) * 128]], axis=0)
        hi = blk2.astype(jnp.bfloat16)
        lo = (blk2 - hi.astype(jnp.float32)).astype(jnp.bfloat16)
        pb = perm.astype(jnp.bfloat16)
        dn = (((1,), (1,)), ((), ()))
        oc = lax.dot_general(
            pb, hi, dimension_numbers=dn,
            preferred_element_type=jnp.float32) + lax.dot_general(
            pb, lo, dimension_numbers=dn,
            preferred_element_type=jnp.float32)
        out_ref[0, j * 64:(j + 1) * 64, 0:_D] = oc[0:_D, 0:_D]
        out_ref[0, j * 64:(j + 1) * 64, _D:2 * _D] = oc[_D:128, 0:_D]
        out_ref[0, (j + 1) * 64:(j + 2) * 64, 0:_D] = oc[0:_D, _D:128]
        out_ref[0, (j + 1) * 64:(j + 2) * 64, _D:2 * _D] = oc[_D:128,
                                                             _D:128]


_ta = pl.pallas_call(
    _ta_body,
    grid=(_T, _NSTEP),
    in_specs=[pl.BlockSpec((1, _D, _W), lambda t, c: (t, 0, c))],
    out_specs=pl.BlockSpec((1, _W // 2, 128), lambda t, c: (t, c, 0)),
    out_shape=jax.ShapeDtypeStruct((_T, _VP2, 2 * _D), jnp.float32),
)


_CHUNK = _B // _NW
_K = 4
_G = 3


def _gbody(idx_hbm, pairs_hbm, out_hbm, idx_v, pidx_v, rows_v, gsem, wsem):
    wid = lax.axis_index("s") * _NC + lax.axis_index("c")
    b0 = wid * _CHUNK
    pltpu.sync_copy(idx_hbm.at[:, pl.ds(b0, _CHUNK)], idx_v)

    def prep(t):
        def pr(i, _):
            v = idx_v[t, pl.ds(i * 16, 16)]
            pidx_v[t & (_K - 1), pl.ds(i * 16, 16)] = (
                lax.shift_right_logical(v, 1))
            return 0

        lax.fori_loop(0, _CHUNK // 16, pr, 0)
        pltpu.async_copy(
            pairs_hbm.at[t].at[pidx_v.at[t & (_K - 1)]],
            rows_v.at[t & (_K - 1)], gsem)

    for t in range(_G):
        prep(t)

    def ch(t, _):
        s = t & (_K - 1)
        pltpu.make_async_copy(
            pairs_hbm.at[0].at[pidx_v.at[s]], rows_v.at[s], gsem).wait()
        pltpu.async_copy(
            rows_v.at[s], out_hbm.at[t, pl.ds(b0, _CHUNK), :], wsem)

        @pl.when(t + _G < _T)
        def _():
            @pl.when(t >= 1)
            def _():
                pltpu.make_async_copy(
                    rows_v.at[(t - 1) & (_K - 1)],
                    out_hbm.at[0, pl.ds(0, _CHUNK), :], wsem).wait()

            prep(t + _G)

        return 0

    lax.fori_loop(0, _T, ch, 0)

    def dr(t, _):
        pltpu.make_async_copy(
            rows_v.at[t & (_K - 1)],
            out_hbm.at[0, pl.ds(0, _CHUNK), :], wsem).wait()
        return 0

    lax.fori_loop(_T - _K, _T, dr, 0)


_mesh = plsc.VectorSubcoreMesh(core_axis_name="c", subcore_axis_name="s")

_gather = pl.kernel(
    _gbody,
    out_type=jax.ShapeDtypeStruct((_T, _B, 2 * _D), jnp.float32),
    mesh=_mesh,
    scratch_types=[
        pltpu.VMEM((_T, _CHUNK), jnp.int32),
        pltpu.VMEM((_K, _CHUNK), jnp.int32),
        pltpu.VMEM((_K, _CHUNK, 2 * _D), jnp.float32),
        pltpu.SemaphoreType.DMA,
        pltpu.SemaphoreType.DMA,
    ],
    compiler_params=pltpu.CompilerParams(
        use_tc_tiling_on_sc=True, needs_layout_passes=False),
)


_CB = 512


def _tc_body(g_ref, idx_ref, out_ref):
    gb = g_ref[0]
    h = idx_ref[0, 0] & 1
    sel = jnp.where(h[:, None] == 1, gb[:, _D:], gb[:, :_D])
    out_ref[0] = jnp.transpose(sel)


_tc = pl.pallas_call(
    _tc_body,
    grid=(_T, _B // _CB),
    in_specs=[
        pl.BlockSpec((1, _CB, 2 * _D), lambda t, c: (t, c, 0)),
        pl.BlockSpec((1, 1, _CB), lambda t, c: (t * (_B // _CB) + c, 0, 0)),
    ],
    out_specs=pl.BlockSpec((1, _D, _CB), lambda t, c: (t, 0, c)),
    out_shape=jax.ShapeDtypeStruct((_T, _D, _B), jnp.float32),
)


@jax.jit
def kernel(inputs, tables):
    tabT = jnp.transpose(tables, (0, 2, 1))
    idx = jnp.transpose(inputs).astype(jnp.int32)
    pairs = _ta(tabT)
    g = _gather(idx, pairs)
    out = _tc(g, idx.reshape(_T * (_B // _CB), 1, _CB))
    return out.transpose(2, 0, 1)
